# Initial kernel scaffold; baseline (speedup 1.0000x reference)
import jax
import jax.numpy as jnp
from jax.experimental import pallas as pl


def kernel(x, adj_matrix, W1, b1, gamma1, beta1, W2, b2, gamma2, beta2):
    raise NotImplementedError("write your pallas kernel here")


# speedup vs baseline: 4037.2549x; 4037.2549x over previous
import jax
import jax.numpy as jnp
from jax.experimental import pallas as pl
from jax.experimental.pallas import tpu as pltpu

_BLK = 256
_EPS = 1e-5


def _gcn_kernel(x_ref, adj_ref, w1_ref, b1_ref, g1_ref, be1_ref,
                w2_ref, b2_ref, g2_ref, be2_ref, out_ref,
                deg_ref, dis_ref, u1_ref, y1_ref, u2_ref, y2_ref):
    p = pl.program_id(0)
    j = pl.program_id(1)
    nblk = pl.num_programs(1)
    rows = pl.ds(j * _BLK, _BLK)

    @pl.when(p == 0)
    def _deg():
        deg_ref[rows, :] = jnp.sum(adj_ref[...], axis=0).reshape(_BLK, 1)

    @pl.when(jnp.logical_and(p == 1, j == 0))
    def _prep1():
        deg = deg_ref[...] + 1.0
        dis = jnp.where(deg > 0, jax.lax.rsqrt(deg), 0.0)
        dis_ref[...] = dis
        h0 = jnp.dot(x_ref[...], w1_ref[...],
                     preferred_element_type=jnp.float32)
        u1_ref[...] = dis * h0

    @pl.when(p == 1)
    def _prop1():
        t = jax.lax.dot_general(adj_ref[...], u1_ref[...],
                                (((0,), (0,)), ((), ())),
                                preferred_element_type=jnp.float32)
        y1_ref[rows, :] = (dis_ref[rows, :] * (t + u1_ref[rows, :])
                           + b1_ref[...])

    @pl.when(jnp.logical_and(p == 2, j == 0))
    def _prep2():
        y = y1_ref[...]
        m = jnp.mean(y, axis=0, keepdims=True)
        v = jnp.mean((y - m) ** 2, axis=0, keepdims=True)
        y = (y - m) * jax.lax.rsqrt(v + _EPS) * g1_ref[...] + be1_ref[...]
        y = jnp.maximum(y, 0.0)
        h1 = jnp.dot(y, w2_ref[...], preferred_element_type=jnp.float32)
        u2_ref[...] = dis_ref[...] * h1

    @pl.when(p == 2)
    def _prop2():
        t = jax.lax.dot_general(adj_ref[...], u2_ref[...],
                                (((0,), (0,)), ((), ())),
                                preferred_element_type=jnp.float32)
        y2_ref[rows, :] = (dis_ref[rows, :] * (t + u2_ref[rows, :])
                           + b2_ref[...])

    @pl.when(jnp.logical_and(p == 2, j == nblk - 1))
    def _final():
        y = y2_ref[...]
        m = jnp.mean(y, axis=0, keepdims=True)
        v = jnp.mean((y - m) ** 2, axis=0, keepdims=True)
        out_ref[...] = ((y - m) * jax.lax.rsqrt(v + _EPS) * g2_ref[...]
                        + be2_ref[...])


def kernel(x, adj_matrix, W1, b1, gamma1, beta1, W2, b2, gamma2, beta2):
    n, d_in = x.shape
    d_h = W1.shape[1]
    d_out = W2.shape[1]
    nblk = n // _BLK

    b1r = b1.reshape(1, d_h)
    g1r = gamma1.reshape(1, d_h)
    be1r = beta1.reshape(1, d_h)
    b2r = b2.reshape(1, d_out)
    g2r = gamma2.reshape(1, d_out)
    be2r = beta2.reshape(1, d_out)

    return pl.pallas_call(
        _gcn_kernel,
        grid=(3, nblk),
        in_specs=[
            pl.BlockSpec((n, d_in), lambda p, j: (0, 0)),
            pl.BlockSpec((n, _BLK), lambda p, j: (0, j)),
            pl.BlockSpec((d_in, d_h), lambda p, j: (0, 0)),
            pl.BlockSpec((1, d_h), lambda p, j: (0, 0)),
            pl.BlockSpec((1, d_h), lambda p, j: (0, 0)),
            pl.BlockSpec((1, d_h), lambda p, j: (0, 0)),
            pl.BlockSpec((d_h, d_out), lambda p, j: (0, 0)),
            pl.BlockSpec((1, d_out), lambda p, j: (0, 0)),
            pl.BlockSpec((1, d_out), lambda p, j: (0, 0)),
            pl.BlockSpec((1, d_out), lambda p, j: (0, 0)),
        ],
        out_specs=pl.BlockSpec((n, d_out), lambda p, j: (0, 0)),
        out_shape=jax.ShapeDtypeStruct((n, d_out), jnp.float32),
        scratch_shapes=[
            pltpu.VMEM((n, 1), jnp.float32),
            pltpu.VMEM((n, 1), jnp.float32),
            pltpu.VMEM((n, d_h), jnp.float32),
            pltpu.VMEM((n, d_h), jnp.float32),
            pltpu.VMEM((n, d_out), jnp.float32),
            pltpu.VMEM((n, d_out), jnp.float32),
        ],
    )(x, adj_matrix, W1, b1r, g1r, be1r, W2, b2r, g2r, be2r)


# speedup vs baseline: 4537.8455x; 1.1240x over previous
import jax
import jax.numpy as jnp
from jax.experimental import pallas as pl
from jax.experimental.pallas import tpu as pltpu

_BLK = 256
_EPS = 1e-5


def _gcn_kernel(x_ref, adj_ref, w1_ref, b1_ref, g1_ref, be1_ref,
                w2_ref, b2_ref, g2_ref, be2_ref, out_ref,
                deg_ref, dis_ref, u1_ref, y1_ref, u2_ref, y2_ref):
    p = pl.program_id(0)
    j = pl.program_id(1)
    nblk = pl.num_programs(1)
    rows = pl.ds(j * _BLK, _BLK)
    adj_blk = adj_ref[:, rows]

    @pl.when(p == 0)
    def _deg():
        deg_ref[rows, :] = jnp.sum(adj_blk, axis=0).reshape(_BLK, 1)

    @pl.when(jnp.logical_and(p == 1, j == 0))
    def _prep1():
        deg = deg_ref[...] + 1.0
        dis = jnp.where(deg > 0, jax.lax.rsqrt(deg), 0.0)
        dis_ref[...] = dis
        h0 = jnp.dot(x_ref[...], w1_ref[...],
                     preferred_element_type=jnp.float32)
        u1_ref[...] = dis * h0

    @pl.when(p == 1)
    def _prop1():
        t = jax.lax.dot_general(adj_blk, u1_ref[...],
                                (((0,), (0,)), ((), ())),
                                preferred_element_type=jnp.float32)
        y1_ref[rows, :] = (dis_ref[rows, :] * (t + u1_ref[rows, :])
                           + b1_ref[...])

    @pl.when(jnp.logical_and(p == 2, j == 0))
    def _prep2():
        y = y1_ref[...]
        m = jnp.mean(y, axis=0, keepdims=True)
        v = jnp.mean((y - m) ** 2, axis=0, keepdims=True)
        y = (y - m) * jax.lax.rsqrt(v + _EPS) * g1_ref[...] + be1_ref[...]
        y = jnp.maximum(y, 0.0)
        h1 = jnp.dot(y, w2_ref[...], preferred_element_type=jnp.float32)
        u2_ref[...] = dis_ref[...] * h1

    @pl.when(p == 2)
    def _prop2():
        t = jax.lax.dot_general(adj_blk, u2_ref[...],
                                (((0,), (0,)), ((), ())),
                                preferred_element_type=jnp.float32)
        y2_ref[rows, :] = (dis_ref[rows, :] * (t + u2_ref[rows, :])
                           + b2_ref[...])

    @pl.when(jnp.logical_and(p == 2, j == nblk - 1))
    def _final():
        y = y2_ref[...]
        m = jnp.mean(y, axis=0, keepdims=True)
        v = jnp.mean((y - m) ** 2, axis=0, keepdims=True)
        out_ref[...] = ((y - m) * jax.lax.rsqrt(v + _EPS) * g2_ref[...]
                        + be2_ref[...])


def kernel(x, adj_matrix, W1, b1, gamma1, beta1, W2, b2, gamma2, beta2):
    n, d_in = x.shape
    d_h = W1.shape[1]
    d_out = W2.shape[1]
    nblk = n // _BLK

    b1r = b1.reshape(1, d_h)
    g1r = gamma1.reshape(1, d_h)
    be1r = beta1.reshape(1, d_h)
    b2r = b2.reshape(1, d_out)
    g2r = gamma2.reshape(1, d_out)
    be2r = beta2.reshape(1, d_out)

    return pl.pallas_call(
        _gcn_kernel,
        grid=(3, nblk),
        in_specs=[
            pl.BlockSpec((n, d_in), lambda p, j: (0, 0)),
            pl.BlockSpec((n, n), lambda p, j: (0, 0)),
            pl.BlockSpec((d_in, d_h), lambda p, j: (0, 0)),
            pl.BlockSpec((1, d_h), lambda p, j: (0, 0)),
            pl.BlockSpec((1, d_h), lambda p, j: (0, 0)),
            pl.BlockSpec((1, d_h), lambda p, j: (0, 0)),
            pl.BlockSpec((d_h, d_out), lambda p, j: (0, 0)),
            pl.BlockSpec((1, d_out), lambda p, j: (0, 0)),
            pl.BlockSpec((1, d_out), lambda p, j: (0, 0)),
            pl.BlockSpec((1, d_out), lambda p, j: (0, 0)),
        ],
        out_specs=pl.BlockSpec((n, d_out), lambda p, j: (0, 0)),
        out_shape=jax.ShapeDtypeStruct((n, d_out), jnp.float32),
        scratch_shapes=[
            pltpu.VMEM((n, 1), jnp.float32),
            pltpu.VMEM((n, 1), jnp.float32),
            pltpu.VMEM((n, d_h), jnp.float32),
            pltpu.VMEM((n, d_h), jnp.float32),
            pltpu.VMEM((n, d_out), jnp.float32),
            pltpu.VMEM((n, d_out), jnp.float32),
        ],
    )(x, adj_matrix, W1, b1r, g1r, be1r, W2, b2r, g2r, be2r)


# speedup vs baseline: 5700.7108x; 1.2563x over previous
import jax
import jax.numpy as jnp
from jax.experimental import pallas as pl

_EPS = 1e-5


def _canon(lhs, rhs):
    return jax.lax.dot_general(lhs, rhs, (((1,), (0,)), ((), ())),
                               preferred_element_type=jnp.float32)


def _gcn_kernel(x_ref, adj_ref, w1_ref, b1_ref, g1_ref, be1_ref,
                w2_ref, b2_ref, g2_ref, be2_ref, out_ref):
    n = adj_ref.shape[0]

    ones = jnp.ones((8, n), jnp.float32)
    deg = _canon(ones, adj_ref[...])[0:1, :] + 1.0
    dis = jnp.where(deg > 0, jax.lax.rsqrt(deg), 0.0)

    h0t = jax.lax.dot_general(w1_ref[...], x_ref[...],
                              (((0,), (1,)), ((), ())),
                              preferred_element_type=jnp.float32)
    u1 = dis * h0t
    t1 = _canon(u1, adj_ref[...])
    y1 = dis * (t1 + u1) + b1_ref[...]

    m = jnp.mean(y1, axis=1, keepdims=True)
    v = jnp.mean((y1 - m) ** 2, axis=1, keepdims=True)
    y1 = (y1 - m) * jax.lax.rsqrt(v + _EPS) * g1_ref[...] + be1_ref[...]
    y1 = jnp.maximum(y1, 0.0)

    h1t = jax.lax.dot_general(w2_ref[...], y1,
                              (((0,), (0,)), ((), ())),
                              preferred_element_type=jnp.float32)
    u2 = dis * h1t
    t2 = _canon(u2, adj_ref[...])
    y2 = dis * (t2 + u2) + b2_ref[...]

    m = jnp.mean(y2, axis=1, keepdims=True)
    v = jnp.mean((y2 - m) ** 2, axis=1, keepdims=True)
    y2 = (y2 - m) * jax.lax.rsqrt(v + _EPS) * g2_ref[...] + be2_ref[...]

    out_ref[...] = y2.T


def kernel(x, adj_matrix, W1, b1, gamma1, beta1, W2, b2, gamma2, beta2):
    n, d_in = x.shape
    d_h = W1.shape[1]
    d_out = W2.shape[1]

    b1c = b1.reshape(d_h, 1)
    g1c = gamma1.reshape(d_h, 1)
    be1c = beta1.reshape(d_h, 1)
    b2c = b2.reshape(d_out, 1)
    g2c = gamma2.reshape(d_out, 1)
    be2c = beta2.reshape(d_out, 1)

    return pl.pallas_call(
        _gcn_kernel,
        out_shape=jax.ShapeDtypeStruct((n, d_out), jnp.float32),
    )(x, adj_matrix, W1, b1c, g1c, be1c, W2, b2c, g2c, be2c)


# speedup vs baseline: 5751.1851x; 1.0089x over previous
import jax
import jax.numpy as jnp
from jax.experimental import pallas as pl
from jax.experimental.pallas import tpu as pltpu

_EPS = 1e-5
_NB = 8


def _canon(lhs, rhs):
    return jax.lax.dot_general(lhs, rhs, (((1,), (0,)), ((), ())),
                               preferred_element_type=jnp.float32)


def _gcn_kernel(x_ref, adj_hbm, w1_ref, b1_ref, g1_ref, be1_ref,
                w2_ref, b2_ref, g2_ref, be2_ref, out_ref,
                adj_vmem, sem):
    n = adj_vmem.shape[0]
    blk = n // _NB

    copies = [
        pltpu.make_async_copy(
            adj_hbm.at[pl.ds(j * blk, blk), :],
            adj_vmem.at[pl.ds(j * blk, blk), :],
            sem.at[j],
        )
        for j in range(_NB)
    ]
    for c in copies:
        c.start()

    h0t = jax.lax.dot_general(w1_ref[...], x_ref[...],
                              (((0,), (1,)), ((), ())),
                              preferred_element_type=jnp.float32)

    ones = jnp.ones((8, blk), jnp.float32)
    deg = jnp.ones((1, n), jnp.float32)
    for j in range(_NB):
        copies[j].wait()
        deg = deg + _canon(ones, adj_vmem[pl.ds(j * blk, blk), :])[0:1, :]
    dis = jnp.where(deg > 0, jax.lax.rsqrt(deg), 0.0)

    u1 = dis * h0t
    t1 = _canon(u1, adj_vmem[...])
    y1 = dis * (t1 + u1) + b1_ref[...]

    m = jnp.mean(y1, axis=1, keepdims=True)
    v = jnp.mean((y1 - m) ** 2, axis=1, keepdims=True)
    y1 = (y1 - m) * jax.lax.rsqrt(v + _EPS) * g1_ref[...] + be1_ref[...]
    y1 = jnp.maximum(y1, 0.0)

    h1t = jax.lax.dot_general(w2_ref[...], y1,
                              (((0,), (0,)), ((), ())),
                              preferred_element_type=jnp.float32)
    u2 = dis * h1t
    t2 = _canon(u2, adj_vmem[...])
    y2 = dis * (t2 + u2) + b2_ref[...]

    m = jnp.mean(y2, axis=1, keepdims=True)
    v = jnp.mean((y2 - m) ** 2, axis=1, keepdims=True)
    y2 = (y2 - m) * jax.lax.rsqrt(v + _EPS) * g2_ref[...] + be2_ref[...]

    out_ref[...] = y2.T


def kernel(x, adj_matrix, W1, b1, gamma1, beta1, W2, b2, gamma2, beta2):
    n, d_in = x.shape
    d_h = W1.shape[1]
    d_out = W2.shape[1]

    b1c = b1.reshape(d_h, 1)
    g1c = gamma1.reshape(d_h, 1)
    be1c = beta1.reshape(d_h, 1)
    b2c = b2.reshape(d_out, 1)
    g2c = gamma2.reshape(d_out, 1)
    be2c = beta2.reshape(d_out, 1)

    vmem = pl.BlockSpec(memory_space=pltpu.MemorySpace.VMEM)
    return pl.pallas_call(
        _gcn_kernel,
        in_specs=[
            vmem,
            pl.BlockSpec(memory_space=pltpu.MemorySpace.HBM),
            vmem, vmem, vmem, vmem, vmem, vmem, vmem, vmem,
        ],
        out_specs=vmem,
        out_shape=jax.ShapeDtypeStruct((n, d_out), jnp.float32),
        scratch_shapes=[
            pltpu.VMEM((n, n), jnp.float32),
            pltpu.SemaphoreType.DMA((_NB,)),
        ],
    )(x, adj_matrix, W1, b1c, g1c, be1c, W2, b2c, g2c, be2c)


# speedup vs baseline: 8106.7005x; 1.4096x over previous
import jax
import jax.numpy as jnp
from jax.experimental import pallas as pl
from jax.experimental.pallas import tpu as pltpu

_EPS = 1e-5
_NB = 8


def _canon(lhs, rhs):
    return jax.lax.dot_general(lhs, rhs, (((1,), (0,)), ((), ())),
                               preferred_element_type=jnp.float32)


def _gcn_kernel(x_ref, adj_hbm, w1_ref, g1_ref, be1_ref,
                w2_ref, g2_ref, be2_ref, out_ref, adj_vmem, sem):
    n = adj_vmem.shape[0]
    blk = n // _NB

    copies = [
        pltpu.make_async_copy(
            adj_hbm.at[pl.ds(j * blk, blk), :],
            adj_vmem.at[pl.ds(j * blk, blk), :],
            sem.at[j],
        )
        for j in range(_NB)
    ]
    for c in copies:
        c.start()

    h0t = jax.lax.dot_general(w1_ref[...], x_ref[...],
                              (((0,), (1,)), ((), ())),
                              preferred_element_type=jnp.float32)
    d_h = h0t.shape[0]
    d_out = w2_ref.shape[1]
    g1c = jnp.transpose(g1_ref[...].reshape(1, d_h))
    be1c = jnp.transpose(be1_ref[...].reshape(1, d_h))

    ones = jnp.ones((8, blk), jnp.float32)
    deg = jnp.ones((1, n), jnp.float32)
    for j in range(_NB):
        copies[j].wait()
        deg = deg + _canon(ones, adj_vmem[pl.ds(j * blk, blk), :])[0:1, :]
    dis = jnp.where(deg > 0, jax.lax.rsqrt(deg), 0.0)

    u1 = dis * h0t
    t1 = _canon(u1, adj_vmem[...])
    y1 = dis * (t1 + u1)

    m = jnp.mean(y1, axis=1, keepdims=True)
    v = jnp.mean((y1 - m) ** 2, axis=1, keepdims=True)
    y1 = (y1 - m) * jax.lax.rsqrt(v + _EPS) * g1c + be1c
    y1 = jnp.maximum(y1, 0.0)

    h1t = jax.lax.dot_general(w2_ref[...], y1,
                              (((0,), (0,)), ((), ())),
                              preferred_element_type=jnp.float32)
    u2 = dis * h1t
    t2 = _canon(u2, adj_vmem[...])
    y2 = dis * (t2 + u2)

    m = jnp.mean(y2, axis=1, keepdims=True)
    v = jnp.mean((y2 - m) ** 2, axis=1, keepdims=True)
    y2 = (y2 - m) * jax.lax.rsqrt(v + _EPS)
    out_ref[...] = (y2.T * g2_ref[...].reshape(1, d_out)
                    + be2_ref[...].reshape(1, d_out))


def kernel(x, adj_matrix, W1, b1, gamma1, beta1, W2, b2, gamma2, beta2):
    n = x.shape[0]
    d_out = W2.shape[1]

    vmem = pl.BlockSpec(memory_space=pltpu.MemorySpace.VMEM)
    return pl.pallas_call(
        _gcn_kernel,
        in_specs=[
            vmem,
            pl.BlockSpec(memory_space=pltpu.MemorySpace.HBM),
            vmem, vmem, vmem, vmem, vmem, vmem,
        ],
        out_specs=vmem,
        out_shape=jax.ShapeDtypeStruct((n, d_out), jnp.float32),
        scratch_shapes=[
            pltpu.VMEM((n, n), jnp.float32),
            pltpu.SemaphoreType.DMA((_NB,)),
        ],
    )(x, adj_matrix, W1, gamma1, beta1, W2, gamma2, beta2)
